# baseline (speedup 1.0000x reference)
import jax
import jax.numpy as jnp
from jax import lax
from jax.experimental import pallas as pl
from jax.experimental.pallas import tpu as pltpu
from jax.experimental.pallas import tpu_sc as plsc

N = 10000
E = 320000
D = 128
DE = 16
PK = 128 // DE
E8 = E // PK

NC = 2
NS = 16
NW = NC * NS

CH = 1000
CHR = CH // PK
NCHUNKS = E // CH
KMAX = NCHUNKS // NW
NBUF = 2
UNROLL = 8
NP_ = 10240
ROWS_PER_TILE = NP_ // NS

_mesh = plsc.VectorSubcoreMesh(core_axis_name="c", subcore_axis_name="s")


def _make_sc_edge(has_c: bool, write_e: bool):
    out_type = [jax.ShapeDtypeStruct((NC, NP_, DE), jnp.float32)]
    if write_e:
        out_type.append(jax.ShapeDtypeStruct((E8, 128), jnp.float32))

    scratch = [
        [pltpu.VMEM((CH,), jnp.int32) for _ in range(NBUF)],
        [pltpu.VMEM((CH,), jnp.int32) for _ in range(NBUF)],
        [pltpu.VMEM((CH, DE), jnp.float32) for _ in range(NBUF)],
        [pltpu.VMEM((CH, DE), jnp.float32) for _ in range(NBUF)],
        [pltpu.VMEM((CHR, 128), jnp.float32) for _ in range(NBUF)]
        if has_c else None,
        pltpu.VMEM((CHR, 128), jnp.float32) if write_e else None,
        pltpu.VMEM_SHARED((NP_, DE), jnp.float32),
        [pltpu.SemaphoreType.DMA for _ in range(NBUF)],
        [pltpu.SemaphoreType.DMA for _ in range(NBUF)],
        pltpu.SemaphoreType.DMA,
    ]
    scratch = [x for x in scratch if x is not None]

    def body(*refs):
        it = iter(refs)
        src_h = next(it)
        dst_h = next(it)
        ps_h = next(it)
        pd_h = next(it)
        c_h = next(it) if has_c else None
        aggp_h = next(it)
        e_h = next(it) if write_e else None
        idx_s = next(it)
        idx_d = next(it)
        buf_s = next(it)
        buf_d = next(it)
        buf_c = next(it) if has_c else None
        buf_e = next(it) if write_e else None
        agg_sh = next(it)
        sem_i = next(it)
        sem_g = next(it)
        sem_e = next(it)

        c = lax.axis_index("c")
        s = lax.axis_index("s")
        wid = s * NC + c
        row0 = s * ROWS_PER_TILE

        zv = jnp.zeros((DE,), jnp.float32)

        def zbody(r, _):
            buf_s[0][r] = zv
            return 0

        lax.fori_loop(0, ROWS_PER_TILE, zbody, 0)
        pltpu.sync_copy(buf_s[0].at[pl.ds(0, ROWS_PER_TILE)],
                        agg_sh.at[pl.ds(row0, ROWS_PER_TILE)])
        plsc.subcore_barrier()

        def base(k):
            return (wid + k * NW) * CH

        def start_idx(k):
            b = k % NBUF
            return (
                pltpu.async_copy(src_h.at[pl.ds(base(k), CH)], idx_s[b],
                                 sem_i[b]),
                pltpu.async_copy(dst_h.at[pl.ds(base(k), CH)], idx_d[b],
                                 sem_i[b]),
            )

        def start_gather(k):
            b = k % NBUF
            hs = [pltpu.async_copy(ps_h.at[idx_s[b]], buf_s[b], sem_g[b]),
                  pltpu.async_copy(pd_h.at[idx_d[b]], buf_d[b], sem_g[b])]
            if has_c:
                hs.append(pltpu.async_copy(
                    c_h.at[pl.ds(base(k) // PK, CHR)], buf_c[b], sem_g[b]))
            return hs

        def compute(k):
            b = k % NBUF

            def ebody(r, _):
                for u in range(UNROLL):
                    j = r * UNROLL + u
                    v = buf_s[b][j] + buf_d[b][j]
                    if has_c:
                        v = v + buf_c[b][r, pl.ds(u * DE, DE)]
                    v = jnp.maximum(v, 0.0)
                    buf_s[b][j] = v
                    if write_e:
                        buf_e[r, pl.ds(u * DE, DE)] = v
                return 0

            lax.fori_loop(0, CHR, ebody, 0)

        idx_h = {}
        gat_h = {}
        est_h = None
        idx_h[0] = start_idx(0)
        idx_h[1] = start_idx(1)
        for h in idx_h[0]:
            h.wait()
        gat_h[0] = start_gather(0)
        for k in range(KMAX):
            b = k % NBUF
            if k + 1 < KMAX:
                for h in idx_h[k + 1]:
                    h.wait()
                gat_h[k + 1] = start_gather(k + 1)
            for h in gat_h[k]:
                h.wait()
            if write_e and est_h is not None:
                est_h.wait()
            compute(k)
            if write_e:
                est_h = pltpu.async_copy(
                    buf_e, e_h.at[pl.ds(base(k) // PK, CHR)], sem_e)
            pltpu.sync_copy(buf_s[b], agg_sh.at[idx_d[b]], add=True)
            if k + 2 < KMAX:
                idx_h[k + 2] = start_idx(k + 2)
        if write_e:
            est_h.wait()

        plsc.subcore_barrier()
        pltpu.sync_copy(agg_sh.at[pl.ds(row0, ROWS_PER_TILE)],
                        aggp_h.at[c, pl.ds(row0, ROWS_PER_TILE)])

    return pl.kernel(body, out_type=tuple(out_type), mesh=_mesh,
                     scratch_types=tuple(scratch),
                     compiler_params=pltpu.CompilerParams(
                         use_tc_tiling_on_sc=False))


_sc_edge0 = _make_sc_edge(has_c=False, write_e=True)
_sc_edge1 = _make_sc_edge(has_c=True, write_e=True)
_sc_edge2 = _make_sc_edge(has_c=True, write_e=False)


BN = 2048
BE8 = 8000


def _proj_body(z_ref, ws_ref, wd_ref, be_ref, ps_ref, pd_ref):
    zb = z_ref[...]
    ps_ref[...] = jnp.dot(zb, ws_ref[...], preferred_element_type=jnp.float32)
    pd_ref[...] = (jnp.dot(zb, wd_ref[...], preferred_element_type=jnp.float32)
                   + be_ref[...])


def _proj(z, ws, wd, be):
    return pl.pallas_call(
        _proj_body,
        grid=(NP_ // BN,),
        in_specs=[
            pl.BlockSpec((BN, D), lambda i: (i, 0)),
            pl.BlockSpec((D, DE), lambda i: (0, 0)),
            pl.BlockSpec((D, DE), lambda i: (0, 0)),
            pl.BlockSpec((1, DE), lambda i: (0, 0)),
        ],
        out_specs=[
            pl.BlockSpec((BN, DE), lambda i: (i, 0)),
            pl.BlockSpec((BN, DE), lambda i: (i, 0)),
        ],
        out_shape=[
            jax.ShapeDtypeStruct((NP_, DE), jnp.float32),
            jax.ShapeDtypeStruct((NP_, DE), jnp.float32),
        ],
    )(z, ws, wd, be.reshape(1, DE))


def _make_node(residual: bool, proj: bool):
    def body(*refs):
        it = iter(refs)
        x_ref = next(it)
        aggp_ref = next(it)
        wna_ref = next(it)
        wnb_ref = next(it)
        bn_ref = next(it)
        ws_ref = next(it) if proj else None
        wd_ref = next(it) if proj else None
        be_ref = next(it) if proj else None
        xo_ref = next(it)
        pso_ref = next(it) if proj else None
        pdo_ref = next(it) if proj else None

        x = x_ref[...]
        agg = aggp_ref[0] + aggp_ref[1]
        h = jnp.dot(x, wna_ref[...], preferred_element_type=jnp.float32)
        h = h + jnp.dot(agg, wnb_ref[...], preferred_element_type=jnp.float32)
        h = jnp.maximum(h + bn_ref[...], 0.0)
        if residual:
            h = 0.5 * (x + h)
        xo_ref[...] = h
        if proj:
            pso_ref[...] = jnp.dot(h, ws_ref[...],
                                   preferred_element_type=jnp.float32)
            pdo_ref[...] = (jnp.dot(h, wd_ref[...],
                                    preferred_element_type=jnp.float32)
                            + be_ref[...])

    def call(x, aggp, wna, wnb, bn, *proj_args):
        in_specs = [
            pl.BlockSpec((BN, D), lambda i: (i, 0)),
            pl.BlockSpec((NC, BN, DE), lambda i: (0, i, 0)),
            pl.BlockSpec((D, D), lambda i: (0, 0)),
            pl.BlockSpec((DE, D), lambda i: (0, 0)),
            pl.BlockSpec((1, D), lambda i: (0, 0)),
        ]
        args = [x, aggp, wna, wnb, bn.reshape(1, D)]
        out_specs = [pl.BlockSpec((BN, D), lambda i: (i, 0))]
        out_shape = [jax.ShapeDtypeStruct((NP_, D), jnp.float32)]
        if proj:
            ws, wd, be = proj_args
            in_specs += [
                pl.BlockSpec((D, DE), lambda i: (0, 0)),
                pl.BlockSpec((D, DE), lambda i: (0, 0)),
                pl.BlockSpec((1, DE), lambda i: (0, 0)),
            ]
            args += [ws, wd, be.reshape(1, DE)]
            out_specs += [
                pl.BlockSpec((BN, DE), lambda i: (i, 0)),
                pl.BlockSpec((BN, DE), lambda i: (i, 0)),
            ]
            out_shape += [
                jax.ShapeDtypeStruct((NP_, DE), jnp.float32),
                jax.ShapeDtypeStruct((NP_, DE), jnp.float32),
            ]
        return pl.pallas_call(
            body, grid=(NP_ // BN,), in_specs=in_specs,
            out_specs=out_specs, out_shape=out_shape,
        )(*args)

    return call


_node_final = _make_node(residual=False, proj=False)


def _make_boundary(residual: bool, two_e: bool):
    def body(*refs):
        it = iter(refs)
        x_ref = next(it)
        aggp_ref = next(it)
        wna_ref = next(it)
        wnb_ref = next(it)
        bn_ref = next(it)
        ws_ref = next(it)
        wd_ref = next(it)
        be_ref = next(it)
        e0_ref = next(it)
        e1_ref = next(it) if two_e else None
        wk_ref = next(it)
        xo_ref = next(it)
        pso_ref = next(it)
        pdo_ref = next(it)
        c_ref = next(it)

        x = x_ref[...]
        agg = aggp_ref[0] + aggp_ref[1]
        h = jnp.dot(x, wna_ref[...], preferred_element_type=jnp.float32)
        h = h + jnp.dot(agg, wnb_ref[...], preferred_element_type=jnp.float32)
        h = jnp.maximum(h + bn_ref[...], 0.0)
        if residual:
            h = 0.5 * (x + h)
        xo_ref[...] = h
        pso_ref[...] = jnp.dot(h, ws_ref[...],
                               preferred_element_type=jnp.float32)
        pdo_ref[...] = (jnp.dot(h, wd_ref[...],
                                preferred_element_type=jnp.float32)
                        + be_ref[...])
        ea = e0_ref[...]
        if two_e:
            ea = 0.5 * (ea + e1_ref[...])
        c_ref[...] = jnp.dot(ea, wk_ref[...],
                             preferred_element_type=jnp.float32)

    def call(x, aggp, wna, wnb, bn, ws, wd, be, wk, *es):
        espec = pl.BlockSpec((BE8, 128), lambda i: (i, 0))
        in_specs = [
            pl.BlockSpec((BN, D), lambda i: (i, 0)),
            pl.BlockSpec((NC, BN, DE), lambda i: (0, i, 0)),
            pl.BlockSpec((D, D), lambda i: (0, 0)),
            pl.BlockSpec((DE, D), lambda i: (0, 0)),
            pl.BlockSpec((1, D), lambda i: (0, 0)),
            pl.BlockSpec((D, DE), lambda i: (0, 0)),
            pl.BlockSpec((D, DE), lambda i: (0, 0)),
            pl.BlockSpec((1, DE), lambda i: (0, 0)),
        ] + [espec] * len(es) + [pl.BlockSpec((128, 128), lambda i: (0, 0))]
        out_specs = [
            pl.BlockSpec((BN, D), lambda i: (i, 0)),
            pl.BlockSpec((BN, DE), lambda i: (i, 0)),
            pl.BlockSpec((BN, DE), lambda i: (i, 0)),
            espec,
        ]
        out_shape = [
            jax.ShapeDtypeStruct((NP_, D), jnp.float32),
            jax.ShapeDtypeStruct((NP_, DE), jnp.float32),
            jax.ShapeDtypeStruct((NP_, DE), jnp.float32),
            jax.ShapeDtypeStruct((E8, 128), jnp.float32),
        ]
        return pl.pallas_call(
            body, grid=(NP_ // BN,), in_specs=in_specs,
            out_specs=out_specs, out_shape=out_shape,
        )(x, aggp, wna, wnb, bn.reshape(1, D), ws, wd, be.reshape(1, DE),
          *es, wk)

    return call


_boundary1 = _make_boundary(residual=False, two_e=False)
_boundary2 = _make_boundary(residual=True, two_e=True)


def _edgec1_body(e_ref, w_ref, c_ref):
    c_ref[...] = jnp.dot(e_ref[...], w_ref[...],
                         preferred_element_type=jnp.float32)


def _edgec2_body(e0_ref, e1_ref, w_ref, c_ref):
    ea = 0.5 * (e0_ref[...] + e1_ref[...])
    c_ref[...] = jnp.dot(ea, w_ref[...], preferred_element_type=jnp.float32)


def _edgec(wk, *es):
    body = _edgec1_body if len(es) == 1 else _edgec2_body
    espec = pl.BlockSpec((BE8, 128), lambda i: (i, 0))
    return pl.pallas_call(
        body,
        grid=(E8 // BE8,),
        in_specs=[espec] * len(es) + [pl.BlockSpec((128, 128),
                                                   lambda i: (0, 0))],
        out_specs=espec,
        out_shape=jax.ShapeDtypeStruct((E8, 128), jnp.float32),
    )(*es, wk)


def kernel(edge_index, z, We0, be0, Wn0, bn0, We1, be1, Wn1, bn1,
           We2, be2, Wn2, bn2):
    src = edge_index[0]
    dst = edge_index[1]
    zp = jnp.pad(z, ((0, NP_ - N), (0, 0)))
    eye8 = jnp.eye(PK, dtype=jnp.float32)
    k1 = jnp.kron(eye8, We1[2 * D:])
    k2 = jnp.kron(eye8, We2[2 * D:])

    ps0, pd0 = _proj(zp, We0[:D], We0[D:], be0)
    aggp0, e0 = _sc_edge0(src, dst, ps0, pd0)
    x1, ps1, pd1, c1 = _boundary1(zp, aggp0, Wn0[:D], Wn0[D:], bn0,
                                  We1[:D], We1[D:2 * D], be1, k1, e0)
    aggp1, e1 = _sc_edge1(src, dst, ps1, pd1, c1)
    x2, ps2, pd2, c2 = _boundary2(x1, aggp1, Wn1[:D], Wn1[D:], bn1,
                                  We2[:D], We2[D:2 * D], be2, k2, e0, e1)
    (aggp2,) = _sc_edge2(src, dst, ps2, pd2, c2)
    return _node_final(x2, aggp2, Wn2[:D], Wn2[D:], bn2)[0][:N]


# speedup vs baseline: 1.0329x; 1.0329x over previous
import jax
import jax.numpy as jnp
from jax import lax
from jax.experimental import pallas as pl
from jax.experimental.pallas import tpu as pltpu
from jax.experimental.pallas import tpu_sc as plsc

N = 10000
E = 320000
D = 128
DE = 16
PK = 128 // DE
E8 = E // PK

NC = 2
NS = 16
NW = NC * NS

CH = 1000
CHR = CH // PK
NCHUNKS = E // CH
KMAX = NCHUNKS // NW
NBUF = 2
UNROLL = 8
NP_ = 10240
ROWS_PER_TILE = NP_ // NS

_mesh = plsc.VectorSubcoreMesh(core_axis_name="c", subcore_axis_name="s")


def _make_sc_edge(has_c: bool, write_e: bool):
    out_type = [jax.ShapeDtypeStruct((NC, NP_, DE), jnp.float32)]
    if write_e:
        out_type.append(jax.ShapeDtypeStruct((E8, 128), jnp.float32))

    scratch = [
        [pltpu.VMEM((CH,), jnp.int32) for _ in range(NBUF)],
        [pltpu.VMEM((CH,), jnp.int32) for _ in range(NBUF)],
        [pltpu.VMEM((CH, DE), jnp.float32) for _ in range(NBUF)],
        [pltpu.VMEM((CH, DE), jnp.float32) for _ in range(NBUF)],
        [pltpu.VMEM((CHR, 128), jnp.float32) for _ in range(NBUF)]
        if has_c else None,
        pltpu.VMEM((CHR, 128), jnp.float32) if write_e else None,
        pltpu.VMEM_SHARED((NP_, DE), jnp.float32),
        [pltpu.SemaphoreType.DMA for _ in range(NBUF)],
        [pltpu.SemaphoreType.DMA for _ in range(NBUF)],
        pltpu.SemaphoreType.DMA,
    ]
    scratch = [x for x in scratch if x is not None]

    def body(*refs):
        it = iter(refs)
        src_h = next(it)
        dst_h = next(it)
        ps_h = next(it)
        pd_h = next(it)
        c_h = next(it) if has_c else None
        aggp_h = next(it)
        e_h = next(it) if write_e else None
        idx_s = next(it)
        idx_d = next(it)
        buf_s = next(it)
        buf_d = next(it)
        buf_c = next(it) if has_c else None
        buf_e = next(it) if write_e else None
        agg_sh = next(it)
        sem_i = next(it)
        sem_g = next(it)
        sem_e = next(it)

        c = lax.axis_index("c")
        s = lax.axis_index("s")
        wid = s * NC + c
        row0 = s * ROWS_PER_TILE

        zv = jnp.zeros((DE,), jnp.float32)

        def zbody(r, _):
            buf_s[0][r] = zv
            return 0

        lax.fori_loop(0, ROWS_PER_TILE, zbody, 0)
        pltpu.sync_copy(buf_s[0].at[pl.ds(0, ROWS_PER_TILE)],
                        agg_sh.at[pl.ds(row0, ROWS_PER_TILE)])
        plsc.subcore_barrier()

        def base(k):
            return (wid + k * NW) * CH

        def start_idx(k):
            b = k % NBUF
            return (
                pltpu.async_copy(src_h.at[pl.ds(base(k), CH)], idx_s[b],
                                 sem_i[b]),
                pltpu.async_copy(dst_h.at[pl.ds(base(k), CH)], idx_d[b],
                                 sem_i[b]),
            )

        def start_gather(k):
            b = k % NBUF
            hs = [pltpu.async_copy(ps_h.at[idx_s[b]], buf_s[b], sem_g[b]),
                  pltpu.async_copy(pd_h.at[idx_d[b]], buf_d[b], sem_g[b])]
            if has_c:
                hs.append(pltpu.async_copy(
                    c_h.at[pl.ds(base(k) // PK, CHR)], buf_c[b], sem_g[b]))
            return hs

        def compute(k):
            b = k % NBUF

            def ebody(r, _):
                for u in range(UNROLL):
                    j = r * UNROLL + u
                    v = buf_s[b][j] + buf_d[b][j]
                    if has_c:
                        v = v + buf_c[b][r, pl.ds(u * DE, DE)]
                    v = jnp.maximum(v, 0.0)
                    buf_s[b][j] = v
                    if write_e:
                        buf_e[r, pl.ds(u * DE, DE)] = v
                return 0

            lax.fori_loop(0, CHR, ebody, 0)

        idx_h = {}
        gat_h = {}
        est_h = None
        idx_h[0] = start_idx(0)
        idx_h[1] = start_idx(1)
        for h in idx_h[0]:
            h.wait()
        gat_h[0] = start_gather(0)
        for k in range(KMAX):
            b = k % NBUF
            if k + 1 < KMAX:
                for h in idx_h[k + 1]:
                    h.wait()
                gat_h[k + 1] = start_gather(k + 1)
            for h in gat_h[k]:
                h.wait()
            if write_e and est_h is not None:
                est_h.wait()
            compute(k)
            if write_e:
                est_h = pltpu.async_copy(
                    buf_e, e_h.at[pl.ds(base(k) // PK, CHR)], sem_e)
            pltpu.sync_copy(buf_s[b], agg_sh.at[idx_d[b]], add=True)
            if k + 2 < KMAX:
                idx_h[k + 2] = start_idx(k + 2)
        if write_e:
            est_h.wait()

        plsc.subcore_barrier()
        pltpu.sync_copy(agg_sh.at[pl.ds(row0, ROWS_PER_TILE)],
                        aggp_h.at[c, pl.ds(row0, ROWS_PER_TILE)])

    return pl.kernel(body, out_type=tuple(out_type), mesh=_mesh,
                     scratch_types=tuple(scratch),
                     compiler_params=pltpu.CompilerParams(
                         use_tc_tiling_on_sc=False))


_sc_edge0 = _make_sc_edge(has_c=False, write_e=True)
_sc_edge1 = _make_sc_edge(has_c=True, write_e=True)
_sc_edge2 = _make_sc_edge(has_c=True, write_e=False)


BN = 2000
BE8 = 8000


def _proj_body(z_ref, ws_ref, wd_ref, be_ref, ps_ref, pd_ref):
    zb = z_ref[...]
    ps_ref[...] = jnp.dot(zb, ws_ref[...], preferred_element_type=jnp.float32)
    pd_ref[...] = (jnp.dot(zb, wd_ref[...], preferred_element_type=jnp.float32)
                   + be_ref[...])


def _proj(z, ws, wd, be):
    return pl.pallas_call(
        _proj_body,
        grid=(N // BN,),
        in_specs=[
            pl.BlockSpec((BN, D), lambda i: (i, 0)),
            pl.BlockSpec((D, DE), lambda i: (0, 0)),
            pl.BlockSpec((D, DE), lambda i: (0, 0)),
            pl.BlockSpec((1, DE), lambda i: (0, 0)),
        ],
        out_specs=[
            pl.BlockSpec((BN, DE), lambda i: (i, 0)),
            pl.BlockSpec((BN, DE), lambda i: (i, 0)),
        ],
        out_shape=[
            jax.ShapeDtypeStruct((N, DE), jnp.float32),
            jax.ShapeDtypeStruct((N, DE), jnp.float32),
        ],
    )(z, ws, wd, be.reshape(1, DE))


def _make_node(residual: bool, proj: bool):
    def body(*refs):
        it = iter(refs)
        x_ref = next(it)
        aggp_ref = next(it)
        wna_ref = next(it)
        wnb_ref = next(it)
        bn_ref = next(it)
        ws_ref = next(it) if proj else None
        wd_ref = next(it) if proj else None
        be_ref = next(it) if proj else None
        xo_ref = next(it)
        pso_ref = next(it) if proj else None
        pdo_ref = next(it) if proj else None

        x = x_ref[...]
        agg = aggp_ref[0] + aggp_ref[1]
        h = jnp.dot(x, wna_ref[...], preferred_element_type=jnp.float32)
        h = h + jnp.dot(agg, wnb_ref[...], preferred_element_type=jnp.float32)
        h = jnp.maximum(h + bn_ref[...], 0.0)
        if residual:
            h = 0.5 * (x + h)
        xo_ref[...] = h
        if proj:
            pso_ref[...] = jnp.dot(h, ws_ref[...],
                                   preferred_element_type=jnp.float32)
            pdo_ref[...] = (jnp.dot(h, wd_ref[...],
                                    preferred_element_type=jnp.float32)
                            + be_ref[...])

    def call(x, aggp, wna, wnb, bn, *proj_args):
        in_specs = [
            pl.BlockSpec((BN, D), lambda i: (i, 0)),
            pl.BlockSpec((NC, BN, DE), lambda i: (0, i, 0)),
            pl.BlockSpec((D, D), lambda i: (0, 0)),
            pl.BlockSpec((DE, D), lambda i: (0, 0)),
            pl.BlockSpec((1, D), lambda i: (0, 0)),
        ]
        args = [x, aggp, wna, wnb, bn.reshape(1, D)]
        out_specs = [pl.BlockSpec((BN, D), lambda i: (i, 0))]
        out_shape = [jax.ShapeDtypeStruct((N, D), jnp.float32)]
        if proj:
            ws, wd, be = proj_args
            in_specs += [
                pl.BlockSpec((D, DE), lambda i: (0, 0)),
                pl.BlockSpec((D, DE), lambda i: (0, 0)),
                pl.BlockSpec((1, DE), lambda i: (0, 0)),
            ]
            args += [ws, wd, be.reshape(1, DE)]
            out_specs += [
                pl.BlockSpec((BN, DE), lambda i: (i, 0)),
                pl.BlockSpec((BN, DE), lambda i: (i, 0)),
            ]
            out_shape += [
                jax.ShapeDtypeStruct((N, DE), jnp.float32),
                jax.ShapeDtypeStruct((N, DE), jnp.float32),
            ]
        return pl.pallas_call(
            body, grid=(N // BN,), in_specs=in_specs,
            out_specs=out_specs, out_shape=out_shape,
        )(*args)

    return call


_node_final = _make_node(residual=False, proj=False)


def _make_boundary(residual: bool, two_e: bool):
    def body(*refs):
        it = iter(refs)
        x_ref = next(it)
        aggp_ref = next(it)
        wna_ref = next(it)
        wnb_ref = next(it)
        bn_ref = next(it)
        ws_ref = next(it)
        wd_ref = next(it)
        be_ref = next(it)
        e0_ref = next(it)
        e1_ref = next(it) if two_e else None
        wk_ref = next(it)
        xo_ref = next(it)
        pso_ref = next(it)
        pdo_ref = next(it)
        c_ref = next(it)

        x = x_ref[...]
        agg = aggp_ref[0] + aggp_ref[1]
        h = jnp.dot(x, wna_ref[...], preferred_element_type=jnp.float32)
        h = h + jnp.dot(agg, wnb_ref[...], preferred_element_type=jnp.float32)
        h = jnp.maximum(h + bn_ref[...], 0.0)
        if residual:
            h = 0.5 * (x + h)
        xo_ref[...] = h
        pso_ref[...] = jnp.dot(h, ws_ref[...],
                               preferred_element_type=jnp.float32)
        pdo_ref[...] = (jnp.dot(h, wd_ref[...],
                                preferred_element_type=jnp.float32)
                        + be_ref[...])
        ea = e0_ref[...]
        if two_e:
            ea = 0.5 * (ea + e1_ref[...])
        c_ref[...] = jnp.dot(ea, wk_ref[...],
                             preferred_element_type=jnp.float32)

    def call(x, aggp, wna, wnb, bn, ws, wd, be, wk, *es):
        espec = pl.BlockSpec((BE8, 128), lambda i: (i, 0))
        in_specs = [
            pl.BlockSpec((BN, D), lambda i: (i, 0)),
            pl.BlockSpec((NC, BN, DE), lambda i: (0, i, 0)),
            pl.BlockSpec((D, D), lambda i: (0, 0)),
            pl.BlockSpec((DE, D), lambda i: (0, 0)),
            pl.BlockSpec((1, D), lambda i: (0, 0)),
            pl.BlockSpec((D, DE), lambda i: (0, 0)),
            pl.BlockSpec((D, DE), lambda i: (0, 0)),
            pl.BlockSpec((1, DE), lambda i: (0, 0)),
        ] + [espec] * len(es) + [pl.BlockSpec((128, 128), lambda i: (0, 0))]
        out_specs = [
            pl.BlockSpec((BN, D), lambda i: (i, 0)),
            pl.BlockSpec((BN, DE), lambda i: (i, 0)),
            pl.BlockSpec((BN, DE), lambda i: (i, 0)),
            espec,
        ]
        out_shape = [
            jax.ShapeDtypeStruct((N, D), jnp.float32),
            jax.ShapeDtypeStruct((N, DE), jnp.float32),
            jax.ShapeDtypeStruct((N, DE), jnp.float32),
            jax.ShapeDtypeStruct((E8, 128), jnp.float32),
        ]
        return pl.pallas_call(
            body, grid=(N // BN,), in_specs=in_specs,
            out_specs=out_specs, out_shape=out_shape,
        )(x, aggp, wna, wnb, bn.reshape(1, D), ws, wd, be.reshape(1, DE),
          *es, wk)

    return call


_boundary1 = _make_boundary(residual=False, two_e=False)
_boundary2 = _make_boundary(residual=True, two_e=True)


def kernel(edge_index, z, We0, be0, Wn0, bn0, We1, be1, Wn1, bn1,
           We2, be2, Wn2, bn2):
    src = edge_index[0]
    dst = edge_index[1]
    eye8 = jnp.eye(PK, dtype=jnp.float32)
    k1 = jnp.kron(eye8, We1[2 * D:])
    k2 = jnp.kron(eye8, We2[2 * D:])

    ps0, pd0 = _proj(z, We0[:D], We0[D:], be0)
    aggp0, e0 = _sc_edge0(src, dst, ps0, pd0)
    x1, ps1, pd1, c1 = _boundary1(z, aggp0, Wn0[:D], Wn0[D:], bn0,
                                  We1[:D], We1[D:2 * D], be1, k1, e0)
    aggp1, e1 = _sc_edge1(src, dst, ps1, pd1, c1)
    x2, ps2, pd2, c2 = _boundary2(x1, aggp1, Wn1[:D], Wn1[D:], bn1,
                                  We2[:D], We2[D:2 * D], be2, k2, e0, e1)
    (aggp2,) = _sc_edge2(src, dst, ps2, pd2, c2)
    return _node_final(x2, aggp2, Wn2[:D], Wn2[D:], bn2)[0]
